# baseline (speedup 1.0000x reference)
import functools

import jax
import jax.numpy as jnp
from jax import lax
from jax.experimental import pallas as pl
from jax.experimental.pallas import tpu as pltpu
from jax.experimental.pallas import tpu_sc as plsc

B = 1024
CTX = 20
V = 100000
D = 64

NC = 2
NS = 16
NW = NC * NS
BPW = B // NW
IPW = BPW * CTX
ICH = 128
NCH = IPW // ICH

VT = 512
NV = (V + VT - 1) // VT


def _pool_body(idx_hbm, table_hbm, out_hbm, idx_v, rows_v, acc_v, sem):
    wid = lax.axis_index("s") * NC + lax.axis_index("c")
    pltpu.sync_copy(idx_hbm.at[wid], idx_v)
    cps = [
        pltpu.async_copy(
            table_hbm.at[idx_v.at[j]],
            rows_v.at[pl.ds(j * ICH, ICH)],
            sem,
        )
        for j in range(NCH)
    ]
    for cp in cps:
        cp.wait()

    inv_ctx = jnp.float32(1.0 / CTX)

    def body(b, carry):
        for dc in range(D // 16):
            s = jnp.zeros((16,), jnp.float32)
            for c in range(CTX):
                s = s + rows_v[b * CTX + c, pl.ds(dc * 16, 16)]
            acc_v[b, pl.ds(dc * 16, 16)] = s * inv_ctx
        return carry

    lax.fori_loop(0, BPW, body, 0)
    pltpu.sync_copy(acc_v, out_hbm.at[pl.ds(wid * BPW, BPW)])


def _make_pooler():
    mesh = plsc.VectorSubcoreMesh(
        core_axis_name="c", subcore_axis_name="s", num_cores=NC, num_subcores=NS
    )
    return functools.partial(
        pl.kernel,
        mesh=mesh,
        out_type=jax.ShapeDtypeStruct((B, D), jnp.float32),
        scratch_types=[
            pltpu.VMEM((NCH, ICH), jnp.int32),
            pltpu.VMEM((IPW, D), jnp.float32),
            pltpu.VMEM((BPW, D), jnp.float32),
            pltpu.SemaphoreType.DMA,
        ],
        compiler_params=pltpu.CompilerParams(use_tc_tiling_on_sc=False),
    )(_pool_body)


def _stats_body(pooled_ref, w_ref, b_ref, lse_ref, m_s, s_s):
    v = pl.program_id(0)

    @pl.when(v == 0)
    def _init():
        m_s[...] = jnp.full((B, 1), -jnp.inf, jnp.float32)
        s_s[...] = jnp.zeros((B, 1), jnp.float32)

    logits = lax.dot_general(
        pooled_ref[...], w_ref[...],
        (((1,), (1,)), ((), ())),
        preferred_element_type=jnp.float32,
    ) + b_ref[...]
    col = v * VT + lax.broadcasted_iota(jnp.int32, (B, VT), 1)
    logits = jnp.where(col < V, logits, -jnp.inf)

    m_old = m_s[...]
    m_new = jnp.maximum(m_old, jnp.max(logits, axis=1, keepdims=True))
    s_new = s_s[...] * jnp.exp(m_old - m_new) + jnp.sum(
        jnp.exp(logits - m_new), axis=1, keepdims=True
    )
    m_s[...] = m_new
    s_s[...] = s_new

    @pl.when(v == NV - 1)
    def _fin():
        lse_ref[...] = m_new + jnp.log(s_new)


def _out_body(pooled_ref, w_ref, b_ref, lse_ref, out_ref):
    logits = lax.dot_general(
        pooled_ref[...], w_ref[...],
        (((1,), (1,)), ((), ())),
        preferred_element_type=jnp.float32,
    )
    out_ref[...] = logits + b_ref[...] - lse_ref[...]


def _make_tc_calls(interpret=False):
    stats = pl.pallas_call(
        _stats_body,
        grid=(NV,),
        in_specs=[
            pl.BlockSpec((B, D), lambda v: (0, 0)),
            pl.BlockSpec((VT, D), lambda v: (v, 0)),
            pl.BlockSpec((1, VT), lambda v: (0, v)),
        ],
        out_specs=pl.BlockSpec((B, 1), lambda v: (0, 0)),
        out_shape=jax.ShapeDtypeStruct((B, 1), jnp.float32),
        scratch_shapes=[
            pltpu.VMEM((B, 1), jnp.float32),
            pltpu.VMEM((B, 1), jnp.float32),
        ],
        interpret=interpret,
    )
    final = pl.pallas_call(
        _out_body,
        grid=(NV,),
        in_specs=[
            pl.BlockSpec((B, D), lambda v: (0, 0)),
            pl.BlockSpec((VT, D), lambda v: (v, 0)),
            pl.BlockSpec((1, VT), lambda v: (0, v)),
            pl.BlockSpec((B, 1), lambda v: (0, 0)),
        ],
        out_specs=pl.BlockSpec((B, VT), lambda v: (0, v)),
        out_shape=jax.ShapeDtypeStruct((B, V), jnp.float32),
        interpret=interpret,
    )
    return stats, final


def kernel(inputs, embed_table, W, b):
    idx3d = inputs.astype(jnp.int32).reshape(NW, NCH, ICH)
    pooled = _make_pooler()(idx3d, embed_table)
    b2d = b.reshape(1, V)
    stats, final = _make_tc_calls()
    lse = stats(pooled, W, b2d)
    return final(pooled, W, b2d, lse)


# speedup vs baseline: 1.4600x; 1.4600x over previous
import functools

import jax
import jax.numpy as jnp
from jax import lax
from jax.experimental import pallas as pl
from jax.experimental.pallas import tpu as pltpu
from jax.experimental.pallas import tpu_sc as plsc

B = 1024
CTX = 20
V = 100000
D = 64

NC = 2
NS = 16
NW = NC * NS
BPW = B // NW
IPW = BPW * CTX
ICH = 128
NCH = IPW // ICH

VT = 2048
NV = (V + VT - 1) // VT
VP = NV * VT


def _pool_body(idx_hbm, table_hbm, out_hbm, idx_v, rows_v, acc_v, sem):
    wid = lax.axis_index("s") * NC + lax.axis_index("c")
    pltpu.sync_copy(idx_hbm.at[wid], idx_v)
    cps = [
        pltpu.async_copy(
            table_hbm.at[idx_v.at[j]],
            rows_v.at[pl.ds(j * ICH, ICH)],
            sem,
        )
        for j in range(NCH)
    ]
    for cp in cps:
        cp.wait()

    inv_ctx = jnp.float32(1.0 / CTX)

    def body(b, carry):
        for dc in range(D // 16):
            s = jnp.zeros((16,), jnp.float32)
            for c in range(CTX):
                s = s + rows_v[b * CTX + c, pl.ds(dc * 16, 16)]
            acc_v[b, pl.ds(dc * 16, 16)] = s * inv_ctx
        return carry

    lax.fori_loop(0, BPW, body, 0)
    pltpu.sync_copy(acc_v, out_hbm.at[pl.ds(wid * BPW, BPW)])


def _make_pooler():
    mesh = plsc.VectorSubcoreMesh(
        core_axis_name="c", subcore_axis_name="s", num_cores=NC, num_subcores=NS
    )
    return functools.partial(
        pl.kernel,
        mesh=mesh,
        out_type=jax.ShapeDtypeStruct((B, D), jnp.float32),
        scratch_types=[
            pltpu.VMEM((NCH, ICH), jnp.int32),
            pltpu.VMEM((IPW, D), jnp.float32),
            pltpu.VMEM((BPW, D), jnp.float32),
            pltpu.SemaphoreType.DMA,
        ],
        compiler_params=pltpu.CompilerParams(use_tc_tiling_on_sc=False),
    )(_pool_body)


def _stats_body(pooled_ref, w_ref, b_ref, lse_ref, m_s, s_s):
    v = pl.program_id(0)

    @pl.when(v == 0)
    def _init():
        m_s[...] = jnp.full((B, 1), -jnp.float32(1e30), jnp.float32)
        s_s[...] = jnp.zeros((B, 1), jnp.float32)

    logits = lax.dot_general(
        pooled_ref[...], w_ref[...],
        (((1,), (1,)), ((), ())),
        preferred_element_type=jnp.float32,
    ) + b_ref[...]

    m_old = m_s[...]
    m_new = jnp.maximum(m_old, jnp.max(logits, axis=1, keepdims=True))
    s_new = s_s[...] * jnp.exp(m_old - m_new) + jnp.sum(
        jnp.exp(logits - m_new), axis=1, keepdims=True
    )
    m_s[...] = m_new
    s_s[...] = s_new

    @pl.when(v == NV - 1)
    def _fin():
        lse_ref[...] = m_new + jnp.log(s_new)


def _out_body(pooled_ref, w_ref, b_ref, lse_ref, out_ref):
    logits = lax.dot_general(
        pooled_ref[...], w_ref[...],
        (((1,), (1,)), ((), ())),
        preferred_element_type=jnp.float32,
    )
    out_ref[...] = logits + b_ref[...] - lse_ref[...]


def _make_tc_calls(interpret=False):
    stats = pl.pallas_call(
        _stats_body,
        grid=(NV,),
        in_specs=[
            pl.BlockSpec((B, D), lambda v: (0, 0)),
            pl.BlockSpec((VT, D), lambda v: (v, 0)),
            pl.BlockSpec((1, VT), lambda v: (0, v)),
        ],
        out_specs=pl.BlockSpec((B, 1), lambda v: (0, 0)),
        out_shape=jax.ShapeDtypeStruct((B, 1), jnp.float32),
        scratch_shapes=[
            pltpu.VMEM((B, 1), jnp.float32),
            pltpu.VMEM((B, 1), jnp.float32),
        ],
        interpret=interpret,
    )
    final = pl.pallas_call(
        _out_body,
        grid=(NV,),
        in_specs=[
            pl.BlockSpec((B, D), lambda v: (0, 0)),
            pl.BlockSpec((VT, D), lambda v: (v, 0)),
            pl.BlockSpec((1, VT), lambda v: (0, v)),
            pl.BlockSpec((B, 1), lambda v: (0, 0)),
        ],
        out_specs=pl.BlockSpec((B, VT), lambda v: (0, v)),
        out_shape=jax.ShapeDtypeStruct((B, V), jnp.float32),
        interpret=interpret,
    )
    return stats, final


def kernel(inputs, embed_table, W, b):
    idx3d = inputs.astype(jnp.int32).reshape(NW, NCH, ICH)
    pooled = _make_pooler()(idx3d, embed_table)
    w_pad = jnp.pad(W, ((0, VP - V), (0, 0)))
    b2d = jnp.pad(b, (0, VP - V), constant_values=-1e30).reshape(1, VP)
    stats, final = _make_tc_calls()
    lse = stats(pooled, w_pad, b2d)
    return final(pooled, w_pad, b2d, lse)


# speedup vs baseline: 1.5142x; 1.0371x over previous
import functools

import jax
import jax.numpy as jnp
from jax import lax
from jax.experimental import pallas as pl
from jax.experimental.pallas import tpu as pltpu
from jax.experimental.pallas import tpu_sc as plsc

B = 1024
CTX = 20
V = 100000
D = 64
DP = 128

NC = 2
NS = 16
NW = NC * NS
BPW = B // NW
IPW = BPW * CTX
ICH = 128
NCH = IPW // ICH

VT = 2048
NV = (V + VT - 1) // VT
VP = NV * VT


def _pool_body(idx_hbm, table_hbm, out_hbm, idx_v, rows_v, acc_v, sem):
    wid = lax.axis_index("s") * NC + lax.axis_index("c")
    pltpu.sync_copy(idx_hbm.at[wid], idx_v)
    cps = [
        pltpu.async_copy(
            table_hbm.at[idx_v.at[j]],
            rows_v.at[pl.ds(j * ICH, ICH)],
            sem,
        )
        for j in range(NCH)
    ]
    for cp in cps:
        cp.wait()

    inv_ctx = jnp.float32(1.0 / CTX)

    def body(b, carry):
        for dc in range(D // 16):
            s = jnp.zeros((16,), jnp.float32)
            for c in range(CTX):
                s = s + rows_v[b * CTX + c, pl.ds(dc * 16, 16)]
            acc_v[b, pl.ds(dc * 16, 16)] = s * inv_ctx
        return carry

    lax.fori_loop(0, BPW, body, 0)
    pltpu.sync_copy(acc_v, out_hbm.at[pl.ds(wid * BPW, BPW)])


def _make_pooler():
    mesh = plsc.VectorSubcoreMesh(
        core_axis_name="c", subcore_axis_name="s", num_cores=NC, num_subcores=NS
    )
    return functools.partial(
        pl.kernel,
        mesh=mesh,
        out_type=jax.ShapeDtypeStruct((B, D), jnp.float32),
        scratch_types=[
            pltpu.VMEM((NCH, ICH), jnp.int32),
            pltpu.VMEM((IPW, DP), jnp.float32),
            pltpu.VMEM((BPW, D), jnp.float32),
            pltpu.SemaphoreType.DMA,
        ],
    )(_pool_body)


def _stats_body(pooled_ref, w_ref, b_ref, lse_ref, m_s, s_s):
    v = pl.program_id(0)

    @pl.when(v == 0)
    def _init():
        m_s[...] = jnp.full((B, 1), -jnp.float32(1e30), jnp.float32)
        s_s[...] = jnp.zeros((B, 1), jnp.float32)

    logits = lax.dot_general(
        pooled_ref[...], w_ref[...],
        (((1,), (1,)), ((), ())),
        preferred_element_type=jnp.float32,
    ) + b_ref[0]

    def _update(lg):
        m_old = m_s[...]
        m_new = jnp.maximum(m_old, jnp.max(lg, axis=1, keepdims=True))
        s_s[...] = s_s[...] * jnp.exp(m_old - m_new) + jnp.sum(
            jnp.exp(lg - m_new), axis=1, keepdims=True
        )
        m_s[...] = m_new

    @pl.when(v < NV - 1)
    def _mid():
        _update(logits)

    @pl.when(v == NV - 1)
    def _fin():
        col = v * VT + lax.broadcasted_iota(jnp.int32, (B, VT), 1)
        _update(jnp.where(col < V, logits, -jnp.float32(1e30)))
        lse_ref[...] = m_s[...] + jnp.log(s_s[...])


def _out_body(pooled_ref, w_ref, b_ref, lse_ref, out_ref):
    logits = lax.dot_general(
        pooled_ref[...], w_ref[...],
        (((1,), (1,)), ((), ())),
        preferred_element_type=jnp.float32,
    )
    out_ref[...] = logits + (b_ref[0] - lse_ref[...])


def _make_tc_calls(interpret=False):
    stats = pl.pallas_call(
        _stats_body,
        grid=(NV,),
        in_specs=[
            pl.BlockSpec((B, D), lambda v: (0, 0)),
            pl.BlockSpec((VT, D), lambda v: (v, 0)),
            pl.BlockSpec((1, 1, VT), lambda v: (v, 0, 0)),
        ],
        out_specs=pl.BlockSpec((B, 1), lambda v: (0, 0)),
        out_shape=jax.ShapeDtypeStruct((B, 1), jnp.float32),
        scratch_shapes=[
            pltpu.VMEM((B, 1), jnp.float32),
            pltpu.VMEM((B, 1), jnp.float32),
        ],
        interpret=interpret,
    )
    final = pl.pallas_call(
        _out_body,
        grid=(NV,),
        in_specs=[
            pl.BlockSpec((B, D), lambda v: (0, 0)),
            pl.BlockSpec((VT, D), lambda v: (v, 0)),
            pl.BlockSpec((1, 1, VT), lambda v: (v, 0, 0)),
            pl.BlockSpec((B, 1), lambda v: (0, 0)),
        ],
        out_specs=pl.BlockSpec((B, VT), lambda v: (0, v)),
        out_shape=jax.ShapeDtypeStruct((B, V), jnp.float32),
        interpret=interpret,
    )
    return stats, final


def kernel(inputs, embed_table, W, b):
    idx3d = inputs.astype(jnp.int32).reshape(NW, NCH, ICH)
    table_pad = jnp.pad(embed_table, ((0, 0), (0, DP - D)))
    pooled = _make_pooler()(idx3d, table_pad)
    b3d = jnp.pad(b, (0, VP - V)).reshape(NV, 1, VT)
    stats, final = _make_tc_calls()
    lse = stats(pooled, W, b3d)
    return final(pooled, W, b3d, lse)
